# baseline (speedup 1.0000x reference)
import jax
import jax.numpy as jnp
from jax import lax
from jax.experimental import pallas as pl
from jax.experimental.pallas import tpu as pltpu
from jax.experimental.pallas import tpu_sc as plsc

VOCAB = 36
POS = 7
ED = 32
OD = 128
B = 16384
RS = 40

NC = 2
NS = 16
NW = NC * NS
SPW = B // NW
BLK = SPW // 8


def _table_body(emb_ref, w_ref, b_ref, t_ref):
    emb = emb_ref[...]
    for p in range(POS):
        tp = jnp.dot(emb, w_ref[p], preferred_element_type=jnp.float32)
        if p == POS - 1:
            tp = tp + b_ref[...]
        t_ref[pl.ds(p * RS, VOCAB), :] = tp


def _build_table(emb, w3, b2):
    return pl.pallas_call(
        _table_body,
        out_shape=jax.ShapeDtypeStruct((POS * RS, OD), jnp.float32),
    )(emb, w3, b2)


def _sc_body(t_hbm, x_hbm, out_hbm, tv, xv, ov):
    wid = lax.axis_index("s") * NC + lax.axis_index("c")
    base = wid * SPW
    pltpu.sync_copy(t_hbm, tv)
    pltpu.sync_copy(x_hbm.at[pl.ds(base * POS, SPW * POS)],
                    xv.at[pl.ds(0, SPW * POS)])

    lanes = lax.iota(jnp.int32, 16)
    cvs = [((lanes + 16 * k) % POS) * (RS * OD) for k in range(4)]

    @plsc.parallel_loop(0, BLK, 1, unroll=2)
    def block(blk):
        av = []
        for k in range(4):
            xk = xv[pl.ds(blk * (8 * POS) + 16 * k, 16)]
            av.append(xk * OD + cvs[k])
        for s in range(8):
            addrs = [av[(POS * s + p) // 16][(POS * s + p) % 16]
                     for p in range(POS)]
            obase = blk * (8 * OD) + s * OD
            for j in range(OD // 16):
                acc = tv[pl.ds(addrs[0] + j * 16, 16)]
                for p in range(1, POS):
                    acc = acc + tv[pl.ds(addrs[p] + j * 16, 16)]
                ov[pl.ds(obase + j * 16, 16)] = acc

    pltpu.sync_copy(ov, out_hbm.at[pl.ds(base * OD, SPW * OD)])


_sc_call = pl.kernel(
    _sc_body,
    mesh=plsc.VectorSubcoreMesh(core_axis_name="c", subcore_axis_name="s"),
    compiler_params=pltpu.CompilerParams(needs_layout_passes=False),
    out_type=jax.ShapeDtypeStruct((B * OD,), jnp.float32),
    scratch_types=[
        pltpu.VMEM((POS * RS * OD,), jnp.float32),
        pltpu.VMEM((SPW * POS + 16,), jnp.int32),
        pltpu.VMEM((SPW * OD,), jnp.float32),
    ],
)


def kernel(x, emb, W, b):
    t = _build_table(emb, W.reshape(POS, ED, OD), b.reshape(1, OD))
    out_flat = _sc_call(t.reshape(-1), x.reshape(-1).astype(jnp.int32))
    return out_flat.reshape(B, OD)


# speedup vs baseline: 2.1014x; 2.1014x over previous
import jax
import jax.numpy as jnp
from jax import lax
from jax.experimental import pallas as pl
from jax.experimental.pallas import tpu as pltpu
from jax.experimental.pallas import tpu_sc as plsc

VOCAB = 36
POS = 7
ED = 32
OD = 128
B = 16384
RS = 40

NC = 2
NS = 16
NW = NC * NS
SPW = B // NW
BLK = SPW // 8


def _table_body(emb_ref, w_ref, b_ref, t_ref):
    emb = emb_ref[...]
    for p in range(POS):
        tp = jnp.dot(emb, w_ref[p], preferred_element_type=jnp.float32)
        if p == POS - 1:
            tp = tp + b_ref[...]
        t_ref[pl.ds(p * RS, VOCAB), :] = tp


def _build_table(emb, w3, b2):
    return pl.pallas_call(
        _table_body,
        out_shape=jax.ShapeDtypeStruct((POS * RS, OD), jnp.float32),
    )(emb, w3, b2)


def _sc_body(t_hbm, x_hbm, out_hbm, tv, xv, bv, ov):
    wid = lax.axis_index("s") * NC + lax.axis_index("c")
    base = wid * SPW
    pltpu.sync_copy(t_hbm, tv)
    pltpu.sync_copy(x_hbm.at[pl.ds(base * POS, SPW * POS)],
                    xv.at[pl.ds(0, SPW * POS)])

    lanes = lax.iota(jnp.int32, 16)

    @plsc.parallel_loop(0, SPW * POS // 16, 1, unroll=4)
    def addr(i):
        f = lanes + i * 16
        q = (f * 9363) >> 16
        p = f - q * POS
        a = xv[pl.ds(i * 16, 16)] * OD + p * (RS * OD)
        plsc.store_scatter(bv, [q * 8 + p], a)

    @plsc.parallel_loop(0, SPW // 2, 1, unroll=4)
    def pair(s2):
        av = bv[pl.ds(s2 * 16, 16)]
        for h in range(2):
            addrs = [av[h * 8 + p] for p in range(POS)]
            obase = s2 * (2 * OD) + h * OD
            for j in range(OD // 16):
                acc = tv[pl.ds(addrs[0] + j * 16, 16)]
                for p in range(1, POS):
                    acc = acc + tv[pl.ds(addrs[p] + j * 16, 16)]
                ov[pl.ds(obase + j * 16, 16)] = acc

    pltpu.sync_copy(ov, out_hbm.at[pl.ds(base * OD, SPW * OD)])


_sc_call = pl.kernel(
    _sc_body,
    mesh=plsc.VectorSubcoreMesh(core_axis_name="c", subcore_axis_name="s"),
    compiler_params=pltpu.CompilerParams(needs_layout_passes=False),
    out_type=jax.ShapeDtypeStruct((B * OD,), jnp.float32),
    scratch_types=[
        pltpu.VMEM((POS * RS * OD,), jnp.float32),
        pltpu.VMEM((SPW * POS + 16,), jnp.int32),
        pltpu.VMEM((SPW * 8,), jnp.int32),
        pltpu.VMEM((SPW * OD,), jnp.float32),
    ],
)


def kernel(x, emb, W, b):
    t = _build_table(emb, W.reshape(POS, ED, OD), b.reshape(1, OD))
    out_flat = _sc_call(t.reshape(-1), x.reshape(-1).astype(jnp.int32))
    return out_flat.reshape(B, OD)
